# baseline (speedup 1.0000x reference)
import functools

import jax
import jax.numpy as jnp
from jax import lax
from jax.experimental import pallas as pl
from jax.experimental.pallas import tpu as pltpu
from jax.experimental.pallas import tpu_sc as plsc

_INFO = plsc.get_sparse_core_info()
_NC, _NS = _INFO.num_cores, _INFO.num_subcores
_NW = _NC * _NS
_BB = 128


@functools.partial(jax.jit, static_argnums=(3, 4))
def _sc_gather(wt2, pair_t, sel_t, n_seq, d):
    n_b = pair_t.shape[1]
    assert n_b == _NW * _BB and d % 8 == 0

    mesh = plsc.VectorSubcoreMesh(core_axis_name="c", subcore_axis_name="s")

    @functools.partial(
        pl.kernel,
        mesh=mesh,
        out_type=jax.ShapeDtypeStruct((n_seq, d, n_b), jnp.float32),
        scratch_types=[
            pltpu.VMEM((n_seq, _BB), jnp.int32),
            pltpu.VMEM((n_seq, _BB), jnp.int32),
            pltpu.VMEM((2, _BB, 2 * d), jnp.float32),
            pltpu.VMEM((2, d, _BB), jnp.float32),
            pltpu.SemaphoreType.DMA,
            pltpu.SemaphoreType.DMA,
        ],
        compiler_params=pltpu.CompilerParams(
            use_tc_tiling_on_sc=True, needs_layout_passes=False
        ),
    )
    def k(wt_hbm, pair_hbm, sel_hbm, out_hbm, pair_v, sel_v, buf_v, slab_v,
          gsem, ssem):
        w = lax.axis_index("s") * _NC + lax.axis_index("c")
        col0 = w * _BB
        pltpu.sync_copy(pair_hbm.at[:, pl.ds(col0, _BB)], pair_v)
        pltpu.sync_copy(sel_hbm.at[:, pl.ds(col0, _BB)], sel_v)

        row_ids = [lax.iota(jnp.int32, 16) + (16 * g) for g in range(8)]

        def gather_start(s, p):
            pltpu.async_copy(wt_hbm.at[pair_v.at[s]], buf_v.at[p], gsem)

        def gather_wait(p):
            pltpu.make_async_copy(
                wt_hbm.at[pair_v.at[0]], buf_v.at[p], gsem
            ).wait()

        def store_start(s, p):
            pltpu.async_copy(
                slab_v.at[p], out_hbm.at[s, :, pl.ds(col0, _BB)], ssem
            )

        def store_wait(p):
            pltpu.make_async_copy(
                slab_v.at[p], out_hbm.at[0, :, pl.ds(col0, _BB)], ssem
            ).wait()

        def transpose_select(s, p):
            bufp = buf_v.at[p]
            slabp = slab_v.at[p]
            sel_vecs = tuple(
                sel_v[s, pl.ds(16 * g, 16)] for g in range(8)
            )

            def dbody(dk, cols):
                for u in range(4):
                    dd = dk * 4 + u
                    for g in range(8):
                        v = plsc.load_gather(bufp, [row_ids[g], cols[g] + u])
                        slabp[dd, pl.ds(16 * g, 16)] = v
                return tuple(c + 4 for c in cols)

            lax.fori_loop(0, d // 4, dbody, sel_vecs)

        gather_start(0, 0)
        gather_wait(0)
        gather_start(1, 1)
        transpose_select(0, 0)
        gather_start(2, 0)
        store_start(0, 0)
        gather_wait(1)
        transpose_select(1, 1)
        gather_start(3, 1)
        store_start(1, 1)

        def unit_pair(t, carry):
            s0 = t * 2
            for p in range(2):
                s = s0 + p
                gather_wait(p)
                transpose_select(s, p)
                gather_start(s + 2, p)
                store_wait(p)
                store_start(s, p)
            return carry

        lax.fori_loop(1, n_seq // 2 - 1, unit_pair, 0)

        for p in range(2):
            s = n_seq - 2 + p
            gather_wait(p)
            transpose_select(s, p)
            store_wait(p)
            store_start(s, p)
        store_wait(0)
        store_wait(1)

    return k(wt2, pair_t, sel_t)


def kernel(input_, weight):
    b, s = input_.shape
    v, d = weight.shape
    assert b == _NW * _BB and v % 2 == 0
    idx_t = input_.T.astype(jnp.int32)
    pair_t = idx_t >> 1
    sel_t = (idx_t & 1) * d
    wt2 = weight.reshape(v // 2, 2 * d)
    out_phys = _sc_gather(wt2, pair_t, sel_t, s, d)
    return out_phys.transpose(2, 0, 1)


# speedup vs baseline: 1.1410x; 1.1410x over previous
import functools

import jax
import jax.numpy as jnp
from jax import lax
from jax.experimental import pallas as pl
from jax.experimental.pallas import tpu as pltpu
from jax.experimental.pallas import tpu_sc as plsc

_INFO = plsc.get_sparse_core_info()
_NC, _NS = _INFO.num_cores, _INFO.num_subcores
_NW = _NC * _NS
_BB = 128


@functools.partial(jax.jit, static_argnums=(3, 4))
def _sc_gather(wt2, pair_t, sel_t, n_seq, d):
    n_b = pair_t.shape[1]
    assert n_b == _NW * _BB and d % 8 == 0

    mesh = plsc.VectorSubcoreMesh(core_axis_name="c", subcore_axis_name="s")

    @functools.partial(
        pl.kernel,
        mesh=mesh,
        out_type=jax.ShapeDtypeStruct((n_seq, d, n_b), jnp.float32),
        scratch_types=[
            pltpu.VMEM((n_seq, _BB), jnp.int32),
            pltpu.VMEM((n_seq, _BB), jnp.int32),
            pltpu.VMEM((2, _BB, 2 * d), jnp.float32),
            pltpu.VMEM((2, d, _BB), jnp.float32),
            pltpu.SemaphoreType.DMA,
            pltpu.SemaphoreType.DMA,
        ],
        compiler_params=pltpu.CompilerParams(
            use_tc_tiling_on_sc=True, needs_layout_passes=False
        ),
    )
    def k(wt_hbm, pair_hbm, sel_hbm, out_hbm, pair_v, sel_v, buf_v, slab_v,
          gsem, ssem):
        w = lax.axis_index("s") * _NC + lax.axis_index("c")
        col0 = w * _BB
        pltpu.sync_copy(pair_hbm.at[:, pl.ds(col0, _BB)], pair_v)
        pltpu.sync_copy(sel_hbm.at[:, pl.ds(col0, _BB)], sel_v)

        row_ids = [lax.iota(jnp.int32, 16) + (16 * g) for g in range(8)]

        def gather_start(s, p):
            pltpu.async_copy(wt_hbm.at[pair_v.at[s]], buf_v.at[p], gsem)

        def gather_wait(p):
            pltpu.make_async_copy(
                wt_hbm.at[pair_v.at[0]], buf_v.at[p], gsem
            ).wait()

        def store_start(s, p):
            pltpu.async_copy(
                slab_v.at[p], out_hbm.at[s, :, pl.ds(col0, _BB)], ssem
            )

        def store_wait(p):
            pltpu.make_async_copy(
                slab_v.at[p], out_hbm.at[0, :, pl.ds(col0, _BB)], ssem
            ).wait()

        def transpose_select(s, p):
            bufp = buf_v.at[p]
            slabp = slab_v.at[p]
            sel_vecs = tuple(
                sel_v[s, pl.ds(16 * g, 16)] for g in range(8)
            )

            def dbody(dk, cols):
                for u in range(4):
                    dd = dk * 4 + u
                    vals = [
                        plsc.load_gather(bufp, [row_ids[g], cols[g] + u])
                        for g in range(8)
                    ]
                    for g in range(8):
                        slabp[dd, pl.ds(16 * g, 16)] = vals[g]
                return tuple(c + 4 for c in cols)

            lax.fori_loop(0, d // 4, dbody, sel_vecs)

        gather_start(0, 0)
        gather_wait(0)
        gather_start(1, 1)
        transpose_select(0, 0)
        gather_start(2, 0)
        store_start(0, 0)
        gather_wait(1)
        transpose_select(1, 1)
        gather_start(3, 1)
        store_start(1, 1)

        def unit_pair(t, carry):
            s0 = t * 2
            for p in range(2):
                s = s0 + p
                gather_wait(p)
                transpose_select(s, p)
                gather_start(s + 2, p)
                store_wait(p)
                store_start(s, p)
            return carry

        lax.fori_loop(1, n_seq // 2 - 1, unit_pair, 0)

        for p in range(2):
            s = n_seq - 2 + p
            gather_wait(p)
            transpose_select(s, p)
            store_wait(p)
            store_start(s, p)
        store_wait(0)
        store_wait(1)

    return k(wt2, pair_t, sel_t)


def kernel(input_, weight):
    b, s = input_.shape
    v, d = weight.shape
    assert b == _NW * _BB and v % 2 == 0
    idx_t = input_.T.astype(jnp.int32)
    pair_t = idx_t >> 1
    sel_t = (idx_t & 1) * d
    wt2 = weight.reshape(v // 2, 2 * d)
    out_phys = _sc_gather(wt2, pair_t, sel_t, s, d)
    return out_phys.transpose(2, 0, 1)
